# baseline (speedup 1.0000x reference)
import functools
import math

import jax
import jax.numpy as jnp
from jax.experimental import pallas as pl
from jax.experimental.pallas import tpu as pltpu

D_MODEL = 1024
_SCALE = math.sqrt(D_MODEL)
_RPB = 8


def _tc_body(idx_ref, rows_ref, out_ref):
    out_ref[...] = rows_ref[...] * _SCALE


def _make_tc_kernel(B: int):
    grid = (B // _RPB,)

    def row_map(i, idx_ref):
        return (idx_ref[i], 0, 0)

    grid_spec = pltpu.PrefetchScalarGridSpec(
        num_scalar_prefetch=1,
        grid=(B,),
        in_specs=[pl.BlockSpec((1, 1, D_MODEL), row_map)],
        out_specs=pl.BlockSpec((1, 1, D_MODEL), lambda i, idx_ref: (i, 0, 0)),
    )
    return pl.pallas_call(
        _tc_body,
        grid_spec=grid_spec,
        out_shape=jax.ShapeDtypeStruct((B, 1, D_MODEL), jnp.float32),
    )


def kernel(x, table):
    B = x.size
    xf = x.reshape(B).astype(jnp.int32)
    out = _make_tc_kernel(B)(xf, table.reshape(table.shape[0], 1, D_MODEL))
    return out.reshape(x.shape + (D_MODEL,))


# speedup vs baseline: 80.3712x; 80.3712x over previous
import functools
import math

import jax
import jax.numpy as jnp
from jax import lax
from jax.experimental import pallas as pl
from jax.experimental.pallas import tpu as pltpu
from jax.experimental.pallas import tpu_sc as plsc

D_MODEL = 1024
_SCALE = math.sqrt(D_MODEL)
_LANES = 16
_NC = 2
_NS = 16
_NW = _NC * _NS
_C = 32
_NB = 2


def _make_sc_kernel(B: int):
    nch = B // (_NW * _C)
    n_outer = nch // _NB
    mesh = plsc.VectorSubcoreMesh(core_axis_name="c", subcore_axis_name="s")

    @functools.partial(
        pl.kernel,
        mesh=mesh,
        out_type=jax.ShapeDtypeStruct((B, D_MODEL), jnp.float32),
        scratch_types=[
            pltpu.VMEM((nch, _C), jnp.int32),
        ]
        + [pltpu.VMEM((_C, D_MODEL), jnp.float32)] * _NB
        + [pltpu.SemaphoreType.DMA] * (2 * _NB),
    )
    def gather_scale(x_hbm, table_hbm, out_hbm, idx_v, *rest):
        bufs = rest[:_NB]
        gsems = rest[_NB:2 * _NB]
        ssems = rest[2 * _NB:]
        wid = lax.axis_index("s") * _NC + lax.axis_index("c")
        base = wid * (nch * _C)
        pltpu.sync_copy(x_hbm.at[wid], idx_v)

        def start_gather(k, b):
            pltpu.async_copy(table_hbm.at[idx_v.at[k]], bufs[b], gsems[b])

        def wait_gather(b):
            pltpu.make_async_copy(
                table_hbm.at[idx_v.at[0]], bufs[b], gsems[b]).wait()

        def start_store(k, b):
            pltpu.async_copy(bufs[b], out_hbm.at[pl.ds(base + k * _C, _C)],
                             ssems[b])

        def wait_store(b):
            pltpu.make_async_copy(bufs[b], out_hbm.at[pl.ds(base, _C)],
                                  ssems[b]).wait()

        def scale(b):
            buf = bufs[b]

            def row_body(r, c2):
                for j in range(D_MODEL // _LANES):
                    sl = pl.ds(j * _LANES, _LANES)
                    buf[r, sl] = buf[r, sl] * _SCALE
                return c2

            lax.fori_loop(0, _C, row_body, 0)

        for b in range(_NB):
            start_gather(b, b)

        def outer(g, carry):
            for b in range(_NB):
                k = g * _NB + b
                wait_gather(b)
                scale(b)
                start_store(k, b)
                bp = (b - 1) % _NB
                cond = (g >= 1) if b == 0 else (g < n_outer - 1)

                @pl.when(cond)
                def _():
                    wait_store(bp)
                    start_gather(k + _NB - 1, bp)
            return carry

        lax.fori_loop(0, n_outer, outer, 0)
        for b in range(_NB):
            wait_store(b)

    return gather_scale


def kernel(x, table):
    B = x.size
    xw = x.reshape(_NW, B // (_NW * _C), _C).astype(jnp.int32)
    out = _make_sc_kernel(B)(xw, table)
    return out.reshape(x.shape + (D_MODEL,))


# speedup vs baseline: 121.6782x; 1.5140x over previous
import functools
import math

import jax
import jax.numpy as jnp
from jax import lax
from jax.experimental import pallas as pl
from jax.experimental.pallas import tpu as pltpu
from jax.experimental.pallas import tpu_sc as plsc

D_MODEL = 1024
_SCALE = math.sqrt(D_MODEL)
_LANES = 16
_NC = 2
_NS = 16
_C = 8
_NB = 8
_NCH0 = 144
_NCH1 = 112


def _make_sc_kernel(B: int):
    nch_t = B // _C
    assert _NS * (_NCH0 + _NCH1) == nch_t
    assert _NCH0 % _NB == 0 and _NCH1 % _NB == 0
    mesh = plsc.VectorSubcoreMesh(core_axis_name="c", subcore_axis_name="s")

    @functools.partial(
        pl.kernel,
        mesh=mesh,
        out_type=jax.ShapeDtypeStruct((B, D_MODEL), jnp.float32),
        scratch_types=[
            pltpu.VMEM((max(_NCH0, _NCH1), _C), jnp.int32),
        ]
        + [pltpu.VMEM((_C, D_MODEL), jnp.float32)] * _NB
        + [pltpu.SemaphoreType.DMA] * (2 * _NB),
    )
    def gather_scale(x_hbm, table_hbm, out_hbm, idx_v, *rest):
        bufs = rest[:_NB]
        gsems = rest[_NB:2 * _NB]
        ssems = rest[2 * _NB:]
        cid = lax.axis_index("c")
        sid = lax.axis_index("s")

        def run(nch, chunk_base):
            n_outer = nch // _NB

            def start_gather(k, b):
                pltpu.async_copy(table_hbm.at[idx_v.at[k]], bufs[b], gsems[b])

            def wait_gather(b):
                pltpu.make_async_copy(
                    table_hbm.at[idx_v.at[0]], bufs[b], gsems[b]).wait()

            def start_store(k, b):
                pltpu.async_copy(
                    bufs[b],
                    out_hbm.at[pl.ds((chunk_base + k) * _C, _C)], ssems[b])

            def wait_store(b):
                pltpu.make_async_copy(
                    bufs[b], out_hbm.at[pl.ds(0, _C)], ssems[b]).wait()

            def scale(b):
                buf = bufs[b]

                def row_body(r, c2):
                    for j in range(D_MODEL // _LANES):
                        sl = pl.ds(j * _LANES, _LANES)
                        buf[r, sl] = buf[r, sl] * _SCALE
                    return c2

                lax.fori_loop(0, _C, row_body, 0)

            for b in range(_NB):
                start_gather(b, b)

            def outer(g, carry):
                for b in range(_NB):
                    k = g * _NB + b
                    wait_gather(b)
                    scale(b)
                    start_store(k, b)
                    bp = (b - 1) % _NB
                    cond = (g >= 1) if b == 0 else (g < n_outer - 1)

                    @pl.when(cond)
                    def _():
                        wait_store(bp)
                        start_gather(k + _NB - 1, bp)
                return carry

            lax.fori_loop(0, n_outer, outer, 0)
            for b in range(_NB):
                wait_store(b)

        @pl.when(cid == 0)
        def _():
            pltpu.sync_copy(x_hbm.at[pl.ds(sid * _NCH0, _NCH0)], idx_v)
            run(_NCH0, sid * _NCH0)

        @pl.when(cid == 1)
        def _():
            pltpu.sync_copy(
                x_hbm.at[pl.ds(_NS * _NCH0 + sid * _NCH1, _NCH1)],
                idx_v.at[pl.ds(0, _NCH1)])
            run(_NCH1, _NS * _NCH0 + sid * _NCH1)

    return gather_scale


def kernel(x, table):
    B = x.size
    xw = x.reshape(B // _C, _C).astype(jnp.int32)
    out = _make_sc_kernel(B)(xw, table)
    return out.reshape(x.shape + (D_MODEL,))


# speedup vs baseline: 121.8294x; 1.0012x over previous
import functools
import math

import jax
import jax.numpy as jnp
from jax import lax
from jax.experimental import pallas as pl
from jax.experimental.pallas import tpu as pltpu
from jax.experimental.pallas import tpu_sc as plsc

D_MODEL = 1024
_SCALE = math.sqrt(D_MODEL)
_LANES = 16
_NC = 2
_NS = 16
_C = 8
_NB = 8
_NCH0 = 112
_NCH1 = 144


def _make_sc_kernel(B: int):
    nch_t = B // _C
    assert _NS * (_NCH0 + _NCH1) == nch_t
    assert _NCH0 % _NB == 0 and _NCH1 % _NB == 0
    mesh = plsc.VectorSubcoreMesh(core_axis_name="c", subcore_axis_name="s")

    @functools.partial(
        pl.kernel,
        mesh=mesh,
        out_type=jax.ShapeDtypeStruct((B, D_MODEL), jnp.float32),
        scratch_types=[
            pltpu.VMEM((max(_NCH0, _NCH1), _C), jnp.int32),
        ]
        + [pltpu.VMEM((_C, D_MODEL), jnp.float32)] * _NB
        + [pltpu.SemaphoreType.DMA] * (2 * _NB),
    )
    def gather_scale(x_hbm, table_hbm, out_hbm, idx_v, *rest):
        bufs = rest[:_NB]
        gsems = rest[_NB:2 * _NB]
        ssems = rest[2 * _NB:]
        cid = lax.axis_index("c")
        sid = lax.axis_index("s")

        def run(nch, chunk_base):
            n_outer = nch // _NB

            def start_gather(k, b):
                pltpu.async_copy(table_hbm.at[idx_v.at[k]], bufs[b], gsems[b])

            def wait_gather(b):
                pltpu.make_async_copy(
                    table_hbm.at[idx_v.at[0]], bufs[b], gsems[b]).wait()

            def start_store(k, b):
                pltpu.async_copy(
                    bufs[b],
                    out_hbm.at[pl.ds((chunk_base + k) * _C, _C)], ssems[b])

            def wait_store(b):
                pltpu.make_async_copy(
                    bufs[b], out_hbm.at[pl.ds(0, _C)], ssems[b]).wait()

            def scale(b):
                buf = bufs[b]

                def row_body(r, c2):
                    for j in range(D_MODEL // _LANES):
                        sl = pl.ds(j * _LANES, _LANES)
                        buf[r, sl] = buf[r, sl] * _SCALE
                    return c2

                lax.fori_loop(0, _C, row_body, 0)

            for b in range(_NB):
                start_gather(b, b)

            def outer(g, carry):
                for b in range(_NB):
                    k = g * _NB + b
                    wait_gather(b)
                    scale(b)
                    start_store(k, b)
                    bp = (b - 1) % _NB
                    cond = (g >= 1) if b == 0 else (g < n_outer - 1)

                    @pl.when(cond)
                    def _():
                        wait_store(bp)
                        start_gather(k + _NB - 1, bp)
                return carry

            lax.fori_loop(0, n_outer, outer, 0)
            for b in range(_NB):
                wait_store(b)

        @pl.when(cid == 0)
        def _():
            pltpu.sync_copy(x_hbm.at[pl.ds(sid * _NCH0, _NCH0)],
                            idx_v.at[pl.ds(0, _NCH0)])
            run(_NCH0, sid * _NCH0)

        @pl.when(cid == 1)
        def _():
            pltpu.sync_copy(
                x_hbm.at[pl.ds(_NS * _NCH0 + sid * _NCH1, _NCH1)],
                idx_v.at[pl.ds(0, _NCH1)])
            run(_NCH1, _NS * _NCH0 + sid * _NCH1)

    return gather_scale


def kernel(x, table):
    B = x.size
    xw = x.reshape(B // _C, _C).astype(jnp.int32)
    out = _make_sc_kernel(B)(xw, table)
    return out.reshape(x.shape + (D_MODEL,))


# speedup vs baseline: 125.4934x; 1.0301x over previous
import functools
import math

import jax
import jax.numpy as jnp
from jax import lax
from jax.experimental import pallas as pl
from jax.experimental.pallas import tpu as pltpu
from jax.experimental.pallas import tpu_sc as plsc

D_MODEL = 1024
_SCALE = math.sqrt(D_MODEL)
_LANES = 16
_NC = 2
_NS = 16
_C = 8
_NB = 8
_NCH0 = 128
_NCH1 = 128


def _make_sc_kernel(B: int):
    nch_t = B // _C
    assert _NS * (_NCH0 + _NCH1) == nch_t
    assert _NCH0 % _NB == 0 and _NCH1 % _NB == 0
    mesh = plsc.VectorSubcoreMesh(core_axis_name="c", subcore_axis_name="s")

    @functools.partial(
        pl.kernel,
        mesh=mesh,
        out_type=jax.ShapeDtypeStruct((B, D_MODEL), jnp.float32),
        scratch_types=[
            pltpu.VMEM((max(_NCH0, _NCH1), _C), jnp.int32),
        ]
        + [pltpu.VMEM((_C, D_MODEL), jnp.float32)] * _NB
        + [pltpu.SemaphoreType.DMA] * (2 * _NB),
    )
    def gather_scale(x_hbm, table_hbm, out_hbm, idx_v, *rest):
        bufs = rest[:_NB]
        gsems = rest[_NB:2 * _NB]
        ssems = rest[2 * _NB:]
        cid = lax.axis_index("c")
        sid = lax.axis_index("s")

        nhalf = _NB // 2
        bufs_g = bufs[:nhalf]
        bufs_s = bufs[nhalf:]

        def run(nch, chunk_base):
            n_outer = nch // nhalf

            def start_gather(k, b):
                pltpu.async_copy(
                    table_hbm.at[idx_v.at[k]], bufs_g[b], gsems[b])

            def wait_gather(b):
                pltpu.make_async_copy(
                    table_hbm.at[idx_v.at[0]], bufs_g[b], gsems[b]).wait()

            def start_store(k, b):
                pltpu.async_copy(
                    bufs_s[b],
                    out_hbm.at[pl.ds((chunk_base + k) * _C, _C)], ssems[b])

            def wait_store(b):
                pltpu.make_async_copy(
                    bufs_s[b], out_hbm.at[pl.ds(0, _C)], ssems[b]).wait()

            def scale(b):
                src = bufs_g[b]
                dst = bufs_s[b]

                def row_body(r, c2):
                    for j in range(D_MODEL // _LANES):
                        sl = pl.ds(j * _LANES, _LANES)
                        dst[r, sl] = src[r, sl] * _SCALE
                    return c2

                lax.fori_loop(0, _C, row_body, 0)

            for b in range(nhalf):
                start_gather(b, b)

            def outer(g, carry):
                for b in range(nhalf):
                    k = g * nhalf + b
                    wait_gather(b)

                    @pl.when(g >= 1)
                    def _():
                        wait_store(b)

                    scale(b)

                    @pl.when(g < n_outer - 1)
                    def _():
                        start_gather(k + nhalf, b)

                    start_store(k, b)
                return carry

            lax.fori_loop(0, n_outer, outer, 0)
            for b in range(nhalf):
                wait_store(b)

        @pl.when(cid == 0)
        def _():
            pltpu.sync_copy(x_hbm.at[pl.ds(sid * _NCH0, _NCH0)],
                            idx_v.at[pl.ds(0, _NCH0)])
            run(_NCH0, sid * _NCH0)

        @pl.when(cid == 1)
        def _():
            pltpu.sync_copy(
                x_hbm.at[pl.ds(_NS * _NCH0 + sid * _NCH1, _NCH1)],
                idx_v.at[pl.ds(0, _NCH1)])
            run(_NCH1, _NS * _NCH0 + sid * _NCH1)

    return gather_scale


def kernel(x, table):
    B = x.size
    xw = x.reshape(B // _C, _C).astype(jnp.int32)
    out = _make_sc_kernel(B)(xw, table)
    return out.reshape(x.shape + (D_MODEL,))


# speedup vs baseline: 129.5566x; 1.0324x over previous
import functools
import math

import jax
import jax.numpy as jnp
from jax import lax
from jax.experimental import pallas as pl
from jax.experimental.pallas import tpu as pltpu
from jax.experimental.pallas import tpu_sc as plsc

D_MODEL = 1024
_SCALE = math.sqrt(D_MODEL)
_LANES = 16
_NC = 2
_NS = 16
_NW = _NC * _NS
_C = 8
_NB = 8


def _make_sc_kernel(B: int, n_cols: int):
    rpw = B // _NW
    nch = rpw // _C
    n_outer = nch // _NB
    wpr = n_cols // rpw
    mesh = plsc.VectorSubcoreMesh(core_axis_name="c", subcore_axis_name="s")

    @functools.partial(
        pl.kernel,
        mesh=mesh,
        out_type=jax.ShapeDtypeStruct((B, D_MODEL), jnp.float32),
        scratch_types=[
            pltpu.VMEM((rpw,), jnp.int32),
        ]
        + [pltpu.VMEM((_C, D_MODEL), jnp.float32)] * _NB
        + [pltpu.SemaphoreType.DMA] * (2 * _NB),
    )
    def gather_scale(x_hbm, table_hbm, out_hbm, idx_v, *rest):
        bufs = rest[:_NB]
        gsems = rest[_NB:2 * _NB]
        ssems = rest[2 * _NB:]
        wid = lax.axis_index("s") * _NC + lax.axis_index("c")
        base = wid * rpw
        pltpu.sync_copy(
            x_hbm.at[wid // wpr, pl.ds((wid % wpr) * rpw, rpw)],
            idx_v)

        def start_gather(k, b):
            pltpu.async_copy(table_hbm.at[idx_v.at[pl.ds(k * _C, _C)]], bufs[b], gsems[b])

        def wait_gather(b):
            pltpu.make_async_copy(
                table_hbm.at[idx_v.at[pl.ds(0, _C)]], bufs[b], gsems[b]).wait()

        def start_store(k, b):
            pltpu.async_copy(
                bufs[b], out_hbm.at[pl.ds(base + k * _C, _C)], ssems[b])

        def wait_store(b):
            pltpu.make_async_copy(
                bufs[b], out_hbm.at[pl.ds(0, _C)], ssems[b]).wait()

        def scale(b):
            buf = bufs[b]

            def row_body(r, c2):
                for j in range(D_MODEL // _LANES):
                    sl = pl.ds(j * _LANES, _LANES)
                    buf[r, sl] = buf[r, sl] * _SCALE
                return c2

            lax.fori_loop(0, _C, row_body, 0)

        for b in range(_NB):
            start_gather(b, b)

        def outer(g, carry):
            for b in range(_NB):
                k = g * _NB + b
                wait_gather(b)
                scale(b)
                start_store(k, b)
                bp = (b - 1) % _NB
                cond = (g >= 1) if b == 0 else (g < n_outer - 1)

                @pl.when(cond)
                def _():
                    wait_store(bp)
                    start_gather(k + _NB - 1, bp)
            return carry

        lax.fori_loop(0, n_outer, outer, 0)
        for b in range(_NB):
            wait_store(b)

    return gather_scale


def kernel(x, table):
    B = x.size
    rpw = B // _NW
    assert x.shape[-1] % rpw == 0
    out = _make_sc_kernel(B, x.shape[-1])(x.astype(jnp.int32), table)
    return out.reshape(x.shape + (D_MODEL,))
